# baseline (speedup 1.0000x reference)
import functools

import jax
import jax.numpy as jnp
from jax import lax
from jax.experimental import pallas as pl
from jax.experimental.pallas import tpu as pltpu
from jax.experimental.pallas import tpu_sc as plsc

_DIM = 300
_CHUNK = 64
_NBUF = 4


@functools.partial(jax.jit, static_argnames=("n_rows",))
def _gather(table, idx_flat, n_rows):
    info = plsc.get_sparse_core_info()
    nc, ns = info.num_cores, info.num_subcores
    nw = nc * ns
    chunks_per_w = n_rows // (_CHUNK * nw)
    idx_per_w = n_rows // nw
    n_quad = chunks_per_w // _NBUF
    mesh = plsc.VectorSubcoreMesh(core_axis_name="c", subcore_axis_name="s")

    @functools.partial(
        pl.kernel,
        mesh=mesh,
        out_type=jax.ShapeDtypeStruct((n_rows, _DIM), jnp.float32),
        scratch_types=[
            pltpu.VMEM((idx_per_w,), jnp.int32),
            pltpu.VMEM((_CHUNK, _DIM), jnp.float32),
            pltpu.VMEM((_CHUNK, _DIM), jnp.float32),
            pltpu.VMEM((_CHUNK, _DIM), jnp.float32),
            pltpu.VMEM((_CHUNK, _DIM), jnp.float32),
            pltpu.SemaphoreType.DMA,
            pltpu.SemaphoreType.DMA,
            pltpu.SemaphoreType.DMA,
            pltpu.SemaphoreType.DMA,
            pltpu.SemaphoreType.DMA,
            pltpu.SemaphoreType.DMA,
            pltpu.SemaphoreType.DMA,
            pltpu.SemaphoreType.DMA,
        ],
    )
    def k(table_hbm, idx_hbm, out_hbm, idx_v,
          rows0, rows1, rows2, rows3, g0, g1, g2, g3, w0, w1, w2, w3):
        wid = lax.axis_index("s") * nc + lax.axis_index("c")
        c0 = wid * chunks_per_w
        rows = (rows0, rows1, rows2, rows3)
        sem_g = (g0, g1, g2, g3)
        sem_w = (w0, w1, w2, w3)

        pltpu.sync_copy(idx_hbm.at[pl.ds(wid * idx_per_w, idx_per_w)], idx_v)

        def issue_gathers(c, b):
            base = c * _CHUNK
            for g in range(_CHUNK // 16):
                vec = idx_v[pl.ds(base + g * 16, 16)]
                for l in range(16):
                    pltpu.async_copy(
                        table_hbm.at[pl.ds(vec[l], 1)],
                        rows[b].at[pl.ds(g * 16 + l, 1)],
                        sem_g[b],
                    )

        def drain_gathers(b):
            pltpu.make_async_copy(
                table_hbm.at[pl.ds(0, _CHUNK)], rows[b], sem_g[b]
            ).wait()

        def write_out(c, b):
            base = (c0 + c) * _CHUNK
            pltpu.async_copy(rows[b], out_hbm.at[pl.ds(base, _CHUNK)], sem_w[b])

        def wait_write(b):
            pltpu.make_async_copy(
                rows[b], out_hbm.at[pl.ds(0, _CHUNK)], sem_w[b]
            ).wait()

        def quad(q, carry):
            nonfirst = q > 0
            for b in range(_NBUF):
                c = q * _NBUF + b
                pl.when(nonfirst)(lambda b=b: wait_write(b))
                issue_gathers(c, b)
                pb = (b - 1) % _NBUF
                if b == 0:
                    pl.when(nonfirst)(lambda: drain_gathers(_NBUF - 1))
                    pl.when(nonfirst)(lambda c=c: write_out(c - 1, _NBUF - 1))
                else:
                    drain_gathers(pb)
                    write_out(c - 1, pb)
            return carry

        lax.fori_loop(0, n_quad, quad, 0)
        drain_gathers(_NBUF - 1)
        write_out(chunks_per_w - 1, _NBUF - 1)
        for b in range(_NBUF):
            wait_write(b)

    return k(table, idx_flat)


def kernel(table, idxes):
    b0, b1 = idxes.shape
    n_rows = b0 * b1
    idx_flat = idxes.reshape(n_rows).astype(jnp.int32)
    out = _gather(table, idx_flat, n_rows)
    return out.reshape(b0, b1, _DIM)


# speedup vs baseline: 1.0014x; 1.0014x over previous
import functools

import jax
import jax.numpy as jnp
from jax import lax
from jax.experimental import pallas as pl
from jax.experimental.pallas import tpu as pltpu
from jax.experimental.pallas import tpu_sc as plsc

_DIM = 300
_CHUNK = 64
_NBUF = 4


@functools.partial(jax.jit, static_argnames=("n_rows",))
def _gather(table, idx_flat, n_rows):
    info = plsc.get_sparse_core_info()
    nc, ns = info.num_cores, info.num_subcores
    nw = nc * ns
    chunks_per_w = n_rows // (_CHUNK * nw)
    idx_per_w = n_rows // nw
    n_quad = chunks_per_w // _NBUF
    mesh = plsc.VectorSubcoreMesh(core_axis_name="c", subcore_axis_name="s")

    @functools.partial(
        pl.kernel,
        mesh=mesh,
        out_type=jax.ShapeDtypeStruct((n_rows, _DIM), jnp.float32),
        scratch_types=[
            pltpu.VMEM((idx_per_w,), jnp.int32),
            pltpu.VMEM((_CHUNK, _DIM), jnp.float32),
            pltpu.VMEM((_CHUNK, _DIM), jnp.float32),
            pltpu.VMEM((_CHUNK, _DIM), jnp.float32),
            pltpu.VMEM((_CHUNK, _DIM), jnp.float32),
            pltpu.SemaphoreType.DMA,
            pltpu.SemaphoreType.DMA,
            pltpu.SemaphoreType.DMA,
            pltpu.SemaphoreType.DMA,
            pltpu.SemaphoreType.DMA,
            pltpu.SemaphoreType.DMA,
            pltpu.SemaphoreType.DMA,
            pltpu.SemaphoreType.DMA,
        ],
    )
    def k(table_hbm, idx_hbm, out_hbm, idx_v,
          rows0, rows1, rows2, rows3, g0, g1, g2, g3, w0, w1, w2, w3):
        wid = lax.axis_index("s") * nc + lax.axis_index("c")
        c0 = wid * chunks_per_w
        rows = (rows0, rows1, rows2, rows3)
        sem_g = (g0, g1, g2, g3)
        sem_w = (w0, w1, w2, w3)

        pltpu.sync_copy(idx_hbm.at[pl.ds(wid * idx_per_w, idx_per_w)], idx_v)

        def issue_gathers(c, b):
            def group(g, carry):
                vec = idx_v[pl.ds(c * _CHUNK + g * 16, 16)]
                for l in range(16):
                    pltpu.async_copy(
                        table_hbm.at[pl.ds(vec[l], 1)],
                        rows[b].at[pl.ds(g * 16 + l, 1)],
                        sem_g[b],
                    )
                return carry

            lax.fori_loop(0, _CHUNK // 16, group, 0)

        def drain_gathers(b):
            pltpu.make_async_copy(
                table_hbm.at[pl.ds(0, _CHUNK)], rows[b], sem_g[b]
            ).wait()

        def write_out(c, b):
            base = (c0 + c) * _CHUNK
            pltpu.async_copy(rows[b], out_hbm.at[pl.ds(base, _CHUNK)], sem_w[b])

        def wait_write(b):
            pltpu.make_async_copy(
                rows[b], out_hbm.at[pl.ds(0, _CHUNK)], sem_w[b]
            ).wait()

        def quad(q, carry):
            nonfirst = q > 0
            for b in range(_NBUF):
                c = q * _NBUF + b
                pl.when(nonfirst)(lambda b=b: wait_write(b))
                issue_gathers(c, b)
                pb = (b - 1) % _NBUF
                if b == 0:
                    pl.when(nonfirst)(lambda: drain_gathers(_NBUF - 1))
                    pl.when(nonfirst)(lambda c=c: write_out(c - 1, _NBUF - 1))
                else:
                    drain_gathers(pb)
                    write_out(c - 1, pb)
            return carry

        lax.fori_loop(0, n_quad, quad, 0)
        drain_gathers(_NBUF - 1)
        write_out(chunks_per_w - 1, _NBUF - 1)
        for b in range(_NBUF):
            wait_write(b)

    return k(table, idx_flat)


def kernel(table, idxes):
    b0, b1 = idxes.shape
    n_rows = b0 * b1
    idx_flat = idxes.reshape(n_rows).astype(jnp.int32)
    out = _gather(table, idx_flat, n_rows)
    return out.reshape(b0, b1, _DIM)
